# Initial kernel scaffold; baseline (speedup 1.0000x reference)
import jax
import jax.numpy as jnp
from jax.experimental import pallas as pl


def kernel(indices, weight):
    raise NotImplementedError("write your pallas kernel here")


# speedup vs baseline: 1.5752x; 1.5752x over previous
import functools

import jax
import jax.numpy as jnp
from jax import lax
from jax.experimental import pallas as pl
from jax.experimental.pallas import tpu as pltpu
from jax.experimental.pallas import tpu_sc as plsc

_LANE = 128
_CHUNK = 8


def _build(rows_total, lane, d, nc, ns):
    nw = nc * ns
    rpw = rows_total // nw
    mesh = plsc.VectorSubcoreMesh(core_axis_name="c", subcore_axis_name="s")

    @functools.partial(
        pl.kernel,
        out_type=jax.ShapeDtypeStruct((rows_total, lane, d), jnp.float32),
        mesh=mesh,
        scratch_types=[
            pltpu.VMEM((rpw, lane), jnp.int32),
            pltpu.VMEM((_CHUNK, lane, d), jnp.float32),
            pltpu.SemaphoreType.DMA,
        ],
        compiler_params=pltpu.CompilerParams(use_tc_tiling_on_sc=False),
    )
    def run(idx_hbm, table_hbm, out_hbm, idx_v, rows_v, sem):
        wid = lax.axis_index("s") * nc + lax.axis_index("c")
        base = wid * rpw
        pltpu.sync_copy(idx_hbm.at[pl.ds(base, rpw)], idx_v)

        def chunk(c, carry):
            r0 = c * _CHUNK
            copies = [
                pltpu.async_copy(table_hbm.at[idx_v.at[r0 + j]], rows_v.at[j], sem)
                for j in range(_CHUNK)
            ]
            for cp in copies:
                cp.wait()
            pltpu.sync_copy(rows_v, out_hbm.at[pl.ds(base + r0, _CHUNK)])
            return carry

        lax.fori_loop(0, rpw // _CHUNK, chunk, 0)

    return run


def kernel(indices, weight):
    b, f = indices.shape
    v, d = weight.shape
    n = b * f
    rows_total = n // _LANE
    idx2d = indices.reshape(rows_total, _LANE).astype(jnp.int32)
    info = plsc.get_sparse_core_info()
    run = _build(rows_total, _LANE, d, info.num_cores, info.num_subcores)
    out = run(idx2d, weight)
    return out.reshape(b, f, d)


# speedup vs baseline: 1.5855x; 1.0065x over previous
import functools

import jax
import jax.numpy as jnp
from jax import lax
from jax.experimental import pallas as pl
from jax.experimental.pallas import tpu as pltpu
from jax.experimental.pallas import tpu_sc as plsc

_CH = 1664
_NCH = 8
_NBUF = 2


def _build(n, d, nc, ns):
    nw = nc * ns
    ipw = n // nw
    assert ipw == _CH * _NCH
    mesh = plsc.VectorSubcoreMesh(core_axis_name="c", subcore_axis_name="s")

    @functools.partial(
        pl.kernel,
        out_type=jax.ShapeDtypeStruct((n, d), jnp.float32),
        mesh=mesh,
        scratch_types=[
            pltpu.VMEM((_NCH, _CH), jnp.int32),
            pltpu.VMEM((_NBUF, _CH, d), jnp.float32),
            pltpu.SemaphoreType.DMA,
            pltpu.SemaphoreType.DMA,
            pltpu.SemaphoreType.DMA,
            pltpu.SemaphoreType.DMA,
        ],
        compiler_params=pltpu.CompilerParams(use_tc_tiling_on_sc=False),
    )
    def run(idx_hbm, table_hbm, out_hbm, idx_v, rows_v, g0, g1, w0, w1):
        wid = lax.axis_index("s") * nc + lax.axis_index("c")
        pltpu.sync_copy(idx_hbm.at[pl.ds(wid * _NCH, _NCH)], idx_v)
        base = wid * ipw
        gsem = (g0, g1)
        wsem = (w0, w1)

        def gather(c, b):
            return pltpu.async_copy(
                table_hbm.at[idx_v.at[c]], rows_v.at[b], gsem[b])

        def write(c, b):
            return pltpu.async_copy(
                rows_v.at[b], out_hbm.at[pl.ds(base + c * _CH, _CH)], wsem[b])

        gathers = [None] * _NCH
        writes = [None] * _NCH
        for c in range(_NCH):
            b = c % _NBUF
            if c >= _NBUF:
                writes[c - _NBUF].wait()
            gathers[c] = gather(c, b)
            if c >= 1:
                gathers[c - 1].wait()
                writes[c - 1] = write(c - 1, (c - 1) % _NBUF)
        gathers[_NCH - 1].wait()
        writes[_NCH - 1] = write(_NCH - 1, (_NCH - 1) % _NBUF)
        writes[_NCH - 2].wait()
        writes[_NCH - 1].wait()

    return run


def kernel(indices, weight):
    b, f = indices.shape
    v, d = weight.shape
    n = b * f
    info = plsc.get_sparse_core_info()
    nw = info.num_cores * info.num_subcores
    idx2d = indices.reshape(nw * _NCH, _CH).astype(jnp.int32)
    run = _build(n, d, info.num_cores, info.num_subcores)
    out = run(idx2d, weight)
    return out.reshape(b, f, d)
